# baseline (speedup 1.0000x reference)
import functools

import jax
import jax.numpy as jnp
from jax import lax
from jax.experimental import pallas as pl
from jax.experimental.pallas import tpu as pltpu
from jax.experimental.pallas import tpu_sc as plsc

B = 4096
C = 26
V = 100000
H = 32
NU = 100000

NC = 2
NS = 16
NW = NC * NS
BPW = B // NW

VB = 16384
VCH = -(-V // VB)
VP = VCH * VB


QB = VB // 4


def _tc_repack(x_ref, o_ref):
    for k in range(4):
        o_ref[:, k * H:(k + 1) * H] = x_ref[0, :, k * QB:(k + 1) * QB].T


_mesh = plsc.VectorSubcoreMesh(core_axis_name="c", subcore_axis_name="s")


@functools.partial(
    pl.kernel,
    mesh=_mesh,
    out_type=jax.ShapeDtypeStruct((B, C * H), jnp.float32),
    scratch_types=[
        pltpu.VMEM((BPW,), jnp.int32),
        pltpu.VMEM((C, BPW), jnp.int32),
        pltpu.VMEM((2, BPW, H), jnp.float32),
        pltpu.SemaphoreType.DMA,
        pltpu.SemaphoreType.DMA,
        pltpu.SemaphoreType.DMA,
        pltpu.SemaphoreType.DMA,
    ],
    compiler_params=pltpu.CompilerParams(use_tc_tiling_on_sc=False),
)
def _sc_gather(uids_hbm, attr_t_hbm, rows_hbm, out_hbm,
               uids_v, attrs_v, colbuf, sem1, semg0, semg1, semw):
    wid = lax.axis_index("s") * NC + lax.axis_index("c")
    base = wid * BPW
    pltpu.sync_copy(uids_hbm.at[pl.ds(base, BPW)], uids_v)

    cps = [pltpu.async_copy(attr_t_hbm.at[pl.ds(c * NU, NU)].at[uids_v],
                            attrs_v.at[c], sem1)
           for c in range(C)]
    for cp in cps:
        cp.wait()

    for c in range(C):
        for i in range(BPW // 16):
            sl = pl.ds(i * 16, 16)
            v = attrs_v[c, sl]
            vb_base = v & ~(VB - 1)
            k = (v & (VB - 1)) >> 12
            l = v & (QB - 1)
            attrs_v[c, sl] = c * VP + vb_base + 4 * l + k

    semg = (semg0, semg1)

    def fire(c):
        pltpu.async_copy(rows_hbm.at[attrs_v.at[c]], colbuf.at[c % 2],
                         semg[c % 2])

    def complete(c):
        buf = colbuf.at[c % 2]
        dummy = out_hbm.at[pl.ds(0, BPW), pl.ds(0, H)]
        pltpu.make_async_copy(dummy, buf, semg[c % 2]).wait()
        pltpu.async_copy(buf, out_hbm.at[pl.ds(base, BPW), pl.ds(c * H, H)],
                         semw)

    def drain_writeout(c):
        dummy = out_hbm.at[pl.ds(0, BPW), pl.ds(0, H)]
        pltpu.make_async_copy(dummy, colbuf.at[c % 2], semw).wait()

    fire(0)
    for c in range(1, C):
        if c >= 2:
            drain_writeout(c - 2)
        fire(c)
        complete(c - 1)
    complete(C - 1)
    drain_writeout(C - 2)
    drain_writeout(C - 1)


BB = 512


def _tc_project(g_ref, ue_ref, w_ref, b_ref, o_ref):
    acc = jnp.dot(ue_ref[...], w_ref[0:H, :], preferred_element_type=jnp.float32)
    acc += jnp.dot(g_ref[...], w_ref[H:, :], preferred_element_type=jnp.float32)
    o_ref[...] = acc + b_ref[...]


def kernel(uids, user_embedding, attr_table, embed_tables, W, b):
    attr_t = attr_table.T.reshape(-1)
    emb_t = embed_tables.transpose(0, 2, 1)

    lines = pl.pallas_call(
        _tc_repack,
        grid=(C, VCH),
        in_specs=[pl.BlockSpec((1, H, VB), lambda c, v: (c, 0, v))],
        out_specs=pl.BlockSpec((VB // 4, 4 * H), lambda c, v: (c * VCH + v, 0)),
        out_shape=jax.ShapeDtypeStruct((C * VP // 4, 4 * H), jnp.float32),
    )(emb_t)
    rows = lines.reshape(C * VP, H)

    gathered = _sc_gather(uids, attr_t, rows)

    out = pl.pallas_call(
        _tc_project,
        grid=(B // BB,),
        in_specs=[
            pl.BlockSpec((BB, C * H), lambda i: (i, 0)),
            pl.BlockSpec((BB, H), lambda i: (i, 0)),
            pl.BlockSpec((C * H + H, H), lambda i: (0, 0)),
            pl.BlockSpec((1, H), lambda i: (0, 0)),
        ],
        out_specs=pl.BlockSpec((BB, H), lambda i: (i, 0)),
        out_shape=jax.ShapeDtypeStruct((B, H), jnp.float32),
    )(gathered, user_embedding, W, b.reshape(1, H))
    return out
